# baseline (speedup 1.0000x reference)
import dataclasses
import functools

import jax
import jax.numpy as jnp
from jax import lax
from jax.experimental import pallas as pl
from jax.experimental.pallas import tpu as pltpu
from jax.experimental.pallas import tpu_sc as plsc

F32 = jnp.float32
I32 = jnp.int32

_NW = 32
_GW = 128
_BE = 640
_CHUNK = 4096
_SLOT = 96
_SENTINEL = 1 << 30


def _sc_compiler_params():
    cp = pltpu.CompilerParams()
    if "needs_layout_passes" in pltpu.CompilerParams.__dataclass_fields__:
        cp = dataclasses.replace(cp, needs_layout_passes=False)
    return cp


def _mm(a, b):
    return lax.dot_general(a, b, (((1,), (0,)), ((), ())),
                           preferred_element_type=F32)


def _sc_gather(x, idx0, idx1):
    e_pad = idx0.shape[0]
    d = x.shape[1]
    mesh = plsc.VectorSubcoreMesh(core_axis_name="core",
                                  subcore_axis_name="subcore")
    i0 = idx0.reshape(1, e_pad)
    i1 = idx1.reshape(1, e_pad)
    out_t = jax.ShapeDtypeStruct((e_pad, d), F32)

    @functools.partial(pl.kernel, out_type=(out_t, out_t), mesh=mesh)
    def k(x_hbm, i0_hbm, i1_hbm, o0_hbm, o1_hbm):
        def body(i_v, o_v):
            pltpu.sync_copy(x_hbm.at[i_v.at[0]], o_v)

        for i_hbm, o_hbm in ((i0_hbm, o0_hbm), (i1_hbm, o1_hbm)):
            pltpu.emit_pipeline(
                body,
                grid=(e_pad // _GW,),
                in_specs=[pl.BlockSpec((1, _GW), lambda i: (0, i))],
                out_specs=[pl.BlockSpec((_GW, d), lambda i: (i, 0))],
                core_axis_name=("core", "subcore"),
                dimension_semantics=(pltpu.PARALLEL,),
            )(i_hbm, o_hbm)

    return k(x, i0, i1)


def _sc_scatter_max(msg, dst, n_nodes):
    e_pad, d = msg.shape
    npw = (-(-n_nodes // _NW) + 7) // 8 * 8
    last = n_nodes - (_NW - 1) * npw
    nchunk = e_pad // _CHUNK
    nslot = -(-_CHUNK // _SLOT)
    mesh = plsc.VectorSubcoreMesh(core_axis_name="core",
                                  subcore_axis_name="subcore")

    @functools.partial(
        pl.kernel,
        out_type=jax.ShapeDtypeStruct((n_nodes, d), F32),
        mesh=mesh,
        compiler_params=_sc_compiler_params(),
        scratch_types=[
            pltpu.VMEM((npw, d), F32),
            pltpu.VMEM((_CHUNK,), I32),
            pltpu.VMEM((nslot, _SLOT), I32),
            pltpu.VMEM((nslot, _SLOT), I32),
            pltpu.VMEM((_SLOT, d), F32),
        ],
    )
    def k(msg_hbm, dst_hbm, out_hbm, acc, idxc, eidb, locb, rowb):
        w = lax.axis_index("core") * 16 + lax.axis_index("subcore")
        lo = w * npw
        hi = jnp.minimum(lo + npw, n_nodes)

        @pl.loop(0, npw)
        def _zero_acc(i):
            for c0 in range(0, d, 16):
                acc[i, pl.ds(c0, 16)] = jnp.zeros((16,), F32)

        @pl.loop(0, nslot)
        def _zero_eid(s):
            for c0 in range(0, _SLOT, 16):
                eidb[s, pl.ds(c0, 16)] = jnp.zeros((16,), I32)

        @pl.loop(0, nchunk)
        def _chunk(c):
            pltpu.sync_copy(dst_hbm.at[pl.ds(c * _CHUNK, _CHUNK)], idxc)

            def comp(j, off):
                v = idxc[pl.ds(j * 16, 16)]
                m = (v >= lo) & (v < hi)
                pos = jnp.cumsum(m.astype(I32))
                tgt = off + pos - 1
                eid = c * _CHUNK + j * 16 + lax.iota(I32, 16)
                plsc.store_scatter(eidb, [tgt // _SLOT, tgt % _SLOT], eid,
                                   mask=m)
                plsc.store_scatter(locb, [tgt // _SLOT, tgt % _SLOT], v - lo,
                                   mask=m)
                return off + jnp.max(pos)

            n_match = lax.fori_loop(0, _CHUNK // 16, comp, jnp.int32(0))
            n_g = (n_match + (_SLOT - 1)) // _SLOT

            @pl.loop(0, n_g)
            def _slot(g):
                pltpu.sync_copy(msg_hbm.at[eidb.at[g]], rowb)
                m_g = jnp.minimum(_SLOT, n_match - g * _SLOT)

                @pl.loop(0, _SLOT // 16)
                def _grp(t):
                    base = t * 16
                    locv = locb[g, pl.ds(base, 16)]
                    for i in range(16):
                        @pl.when(base + i < m_g)
                        def _(i=i, base=base, locv=locv):
                            ld = locv[i]
                            for c0 in range(0, d, 16):
                                sl = pl.ds(c0, 16)
                                acc[ld, sl] = jnp.maximum(
                                    acc[ld, sl], rowb[base + i, sl])

        @pl.when(w < _NW - 1)
        def _():
            pltpu.sync_copy(acc, out_hbm.at[pl.ds(lo, npw)])

        @pl.when(w == _NW - 1)
        def _():
            pltpu.sync_copy(acc.at[pl.ds(0, last)],
                            out_hbm.at[pl.ds((_NW - 1) * npw, last)])

    return k(msg, dst)


def _tc_edge(xi, xj, ef, wts, relu_out):
    e_pad, dn = xi.shape
    de = ef.shape[1]
    dh = wts["A1"].shape[1]
    grid = (e_pad // _BE,)

    def body(xi_r, xj_r, e_r, a1, a2, a3, b1e, a4, b2e, wq, bq, we, be,
             wv, bv, ka, kb, b1h, k2, b2i, m4, oe, opr, oxm):
        xi_v = xi_r[...]
        xj_v = xj_r[...]
        e_v = e_r[...]
        h = jnp.maximum(
            _mm(xi_v, a1[...]) + _mm(e_v, a2[...]) + _mm(xj_v, a3[...])
            + b1e[...], 0.0)
        ne = _mm(h, a4[...]) + b2e[...]
        oe[...] = jnp.maximum(ne, 0.0) if relu_out else ne
        q = jnp.maximum(_mm(xi_v, wq[...]) + bq[...], 0.0)
        ep = jnp.maximum(_mm(e_v, we[...]) + be[...], 0.0)
        v = jnp.maximum(_mm(xj_v, wv[...]) + bv[...], 0.0)
        h2 = jnp.maximum(_mm(q, ka[...]) + _mm(ep, kb[...]) + b1h[...], 0.0)
        lg = _mm(h2, k2[...]) + b2i[...]
        mx = jnp.max(lg, axis=1, keepdims=True)
        z = jnp.exp(lg - mx)
        den = _mm(z, m4[...])
        pr = z / den
        opr[...] = pr
        oxm[...] = pr * v

    full = lambda s: pl.BlockSpec(s, lambda i: (0, 0))
    out_sds = (
        jax.ShapeDtypeStruct((e_pad, de), F32),
        jax.ShapeDtypeStruct((e_pad, dn), F32),
        jax.ShapeDtypeStruct((e_pad, dn), F32),
    )
    return pl.pallas_call(
        body,
        grid=grid,
        in_specs=[
            pl.BlockSpec((_BE, dn), lambda i: (i, 0)),
            pl.BlockSpec((_BE, dn), lambda i: (i, 0)),
            pl.BlockSpec((_BE, de), lambda i: (i, 0)),
            full((dn, dh)), full((de, dh)), full((dn, dh)), full((1, dh)),
            full((dh, de)), full((1, de)),
            full((dn, dn)), full((1, dn)),
            full((de, de)), full((1, de)),
            full((dn, dn)), full((1, dn)),
            full((dn, dh)), full((de, dh)), full((1, dh)),
            full((dh, dn)), full((1, dn)),
            full((dn, dn)),
        ],
        out_specs=[
            pl.BlockSpec((_BE, de), lambda i: (i, 0)),
            pl.BlockSpec((_BE, dn), lambda i: (i, 0)),
            pl.BlockSpec((_BE, dn), lambda i: (i, 0)),
        ],
        out_shape=out_sds,
        compiler_params=pltpu.CompilerParams(
            dimension_semantics=("arbitrary",)),
    )(xi, xj, ef, wts["A1"], wts["A2"], wts["A3"], wts["b1e"], wts["A4"],
      wts["b2e"], wts["Wq"], wts["bq"], wts["We"], wts["be"], wts["Wv"],
      wts["bv"], wts["KA"], wts["KB"], wts["b1h"], wts["K2"], wts["b2i"],
      wts["M4"])


def _tc_prop(x, agg, wts, relu_out):
    n, dn = x.shape
    dmid = wts["P1x"].shape[1]
    bn = 2000
    grid = (n // bn,)

    def body(x_r, a_r, p1x, p1a, pb1, p2, pb2, o_r):
        h3 = jnp.maximum(
            _mm(x_r[...], p1x[...]) + _mm(a_r[...], p1a[...]) + pb1[...], 0.0)
        nx = _mm(h3, p2[...]) + pb2[...]
        o_r[...] = jnp.maximum(nx, 0.0) if relu_out else nx

    full = lambda s: pl.BlockSpec(s, lambda i: (0, 0))
    return pl.pallas_call(
        body,
        grid=grid,
        in_specs=[
            pl.BlockSpec((bn, dn), lambda i: (i, 0)),
            pl.BlockSpec((bn, dn), lambda i: (i, 0)),
            full((dn, dmid)), full((dn, dmid)), full((1, dmid)),
            full((dmid, dn)), full((1, dn)),
        ],
        out_specs=pl.BlockSpec((bn, dn), lambda i: (i, 0)),
        out_shape=jax.ShapeDtypeStruct((n, dn), F32),
        compiler_params=pltpu.CompilerParams(
            dimension_semantics=("arbitrary",)),
    )(x, agg, wts["P1x"], wts["P1a"], wts["pb1"], wts["P2"], wts["pb2"])


def _prep_weights(p, dn, de, nh):
    d_n = dn // nh
    d_e = de // nh
    dc = d_n + d_e
    dh = dn + de
    w = {}
    w["A1"] = p["nn_edge_W1"][:, :dn].T
    w["A2"] = p["nn_edge_W1"][:, dn:dn + de].T
    w["A3"] = p["nn_edge_W1"][:, dn + de:].T
    w["b1e"] = p["nn_edge_b1"][None, :]
    w["A4"] = p["nn_edge_W2"].T
    w["b2e"] = p["nn_edge_b2"][None, :]
    w["Wq"] = p["proj_q_W"].T
    w["bq"] = p["proj_q_b"][None, :]
    w["We"] = p["proj_e_W"].T
    w["be"] = p["proj_e_b"][None, :]
    w["Wv"] = p["proj_v_W"].T
    w["bv"] = p["proj_v_b"][None, :]
    w1t = p["att_W1"].T
    j = jnp.arange(dn)
    ka = jnp.zeros((dn, nh * dc), F32)
    ka = ka.at[j[:, None], (j % nh)[:, None] * dc + jnp.arange(dc)[None, :]
               ].set(w1t[j // nh])
    w["KA"] = ka
    je = jnp.arange(de)
    kb = jnp.zeros((de, nh * dc), F32)
    kb = kb.at[je[:, None], (je % nh)[:, None] * dc + jnp.arange(dc)[None, :]
               ].set(w1t[d_n + je // nh])
    w["KB"] = kb
    w["b1h"] = jnp.tile(p["att_b1"], nh)[None, :]
    w2t = p["att_W2"].T
    r = jnp.arange(nh * dc)
    k2 = jnp.zeros((nh * dc, d_n * nh), F32)
    k2 = k2.at[r[:, None], jnp.arange(d_n)[None, :] * nh + (r // dc)[:, None]
               ].set(w2t[r % dc])
    w["K2"] = k2
    w["b2i"] = jnp.repeat(p["att_b2"], nh)[None, :]
    c = jnp.arange(dn)
    w["M4"] = (c[:, None] % nh == c[None, :] % nh).astype(F32)
    w["P1x"] = p["prop_W1"][:, :dn].T
    w["P1a"] = p["prop_W1"][:, dn:].T
    w["pb1"] = p["prop_b1"][None, :]
    w["P2"] = p["prop_W2"].T
    w["pb2"] = p["prop_b2"][None, :]
    return w


def kernel(node_feature, edge_feature, edges_indices, params):
    n, dn = node_feature.shape
    e, de = edge_feature.shape
    nh = 4
    align = 20480
    e_pad = -(-e // align) * align
    pad = e_pad - e

    idx0 = edges_indices[0].astype(I32)
    idx1 = edges_indices[1].astype(I32)
    zpad = jnp.zeros((pad,), I32)
    g0 = jnp.concatenate([idx0, zpad])
    g1 = jnp.concatenate([idx1, zpad])
    dst = jnp.concatenate([idx0, jnp.full((pad,), _SENTINEL, I32)])
    ef = jnp.concatenate([edge_feature,
                          jnp.zeros((pad, de), F32)], axis=0)

    x = node_feature
    probs = []
    for li in range(len(params)):
        w = _prep_weights(params[li], dn, de, nh)
        xi, xj = _sc_gather(x, g0, g1)
        ne, pr, xm = _tc_edge(xi, xj, ef, w, relu_out=(li == 0))
        agg = _sc_scatter_max(xm, dst, n)
        x = _tc_prop(x, agg, w, relu_out=(li == 0))
        probs.append(pr[:e].reshape(e, dn // nh, nh))
        ef = ne
    return x, ef[:e], probs
